# Initial kernel scaffold; baseline (speedup 1.0000x reference)
import jax
import jax.numpy as jnp
from jax.experimental import pallas as pl


def kernel(x, emb_weight):
    raise NotImplementedError("write your pallas kernel here")


# speedup vs baseline: 2.3498x; 2.3498x over previous
import functools

import jax
import jax.numpy as jnp
from jax import lax
from jax.experimental import pallas as pl
from jax.experimental.pallas import tpu as pltpu
from jax.experimental.pallas import tpu_sc as plsc

_NC = 2
_NS = 16
_NW = _NC * _NS
_CHUNK = 128


def _pos_encoding(seq_len, d_model, min_timescale=1.0, max_timescale=10000.0):
    position = jnp.arange(seq_len, dtype=jnp.float32)
    num_timescales = d_model // 2
    log_timescale_increment = jnp.log(
        jnp.float32(max_timescale) / jnp.float32(min_timescale)
    ) / (num_timescales - 1)
    inv_timescales = min_timescale * jnp.exp(
        jnp.arange(num_timescales, dtype=jnp.float32) * -log_timescale_increment
    )
    scaled_time = position[:, None] * inv_timescales[None, :]
    signal = jnp.concatenate([jnp.sin(scaled_time), jnp.cos(scaled_time)], axis=1)
    if d_model % 2:
        signal = jnp.pad(signal, ((0, 0), (0, 1)))
    return signal


@functools.lru_cache(maxsize=None)
def _make_kernel(n_rows, d_model, seq_len):
    assert n_rows % _NW == 0
    bpw = n_rows // _NW
    assert bpw % _CHUNK == 0
    steps = bpw // _CHUNK
    assert d_model % 16 == 0

    mesh = plsc.VectorSubcoreMesh(core_axis_name="c", subcore_axis_name="s")

    @functools.partial(
        pl.kernel,
        out_type=jax.ShapeDtypeStruct((n_rows, d_model), jnp.float32),
        mesh=mesh,
        scratch_types=[
            pltpu.VMEM((bpw,), jnp.int32),
            pltpu.VMEM((2 * seq_len, d_model), jnp.float32),
            pltpu.VMEM((_CHUNK, d_model), jnp.float32),
            pltpu.SemaphoreType.DMA,
        ],
        compiler_params=pltpu.CompilerParams(use_tc_tiling_on_sc=False),
    )
    def k(emb_hbm, idx_hbm, pe_hbm, out_hbm, idx_v, pe_v, rows_v, sem):
        wid = lax.axis_index("s") * _NC + lax.axis_index("c")
        base = wid * bpw
        pltpu.sync_copy(idx_hbm.at[pl.ds(base, bpw)], idx_v)
        pltpu.sync_copy(pe_hbm, pe_v)

        @pl.loop(0, steps)
        def _step(s_):
            off = s_ * _CHUNK
            phase = lax.rem(off, seq_len)
            pltpu.async_copy(
                emb_hbm.at[idx_v.at[pl.ds(off, _CHUNK)]], rows_v, sem
            ).wait()

            @pl.loop(0, _CHUNK)
            def _row(r):
                src = phase + r
                for c4 in range(d_model // 16):
                    sl = pl.ds(c4 * 16, 16)
                    plsc.addupdate(rows_v.at[r, sl], pe_v[src, sl])

            pltpu.sync_copy(rows_v, out_hbm.at[pl.ds(base + off, _CHUNK)])

    return k


def kernel(x, emb_weight):
    batch, seq_len = x.shape
    _, d_model = emb_weight.shape
    pe = _pos_encoding(seq_len, d_model)
    pe2 = jnp.concatenate([pe, pe], axis=0)
    idx = x.reshape(-1).astype(jnp.int32)
    out = _make_kernel(batch * seq_len, d_model, seq_len)(
        emb_weight.astype(jnp.float32), idx, pe2
    )
    return out.reshape(batch, seq_len, d_model)


# speedup vs baseline: 3.0962x; 1.3176x over previous
import functools

import jax
import jax.numpy as jnp
from jax import lax
from jax.experimental import pallas as pl
from jax.experimental.pallas import tpu as pltpu
from jax.experimental.pallas import tpu_sc as plsc

_NC = 2
_NS = 16
_NW = _NC * _NS
_CHUNK = 128
_NBUF = 4
_LOOKAHEAD = 2


def _pos_encoding(seq_len, d_model, min_timescale=1.0, max_timescale=10000.0):
    position = jnp.arange(seq_len, dtype=jnp.float32)
    num_timescales = d_model // 2
    log_timescale_increment = jnp.log(
        jnp.float32(max_timescale) / jnp.float32(min_timescale)
    ) / (num_timescales - 1)
    inv_timescales = min_timescale * jnp.exp(
        jnp.arange(num_timescales, dtype=jnp.float32) * -log_timescale_increment
    )
    scaled_time = position[:, None] * inv_timescales[None, :]
    signal = jnp.concatenate([jnp.sin(scaled_time), jnp.cos(scaled_time)], axis=1)
    if d_model % 2:
        signal = jnp.pad(signal, ((0, 0), (0, 1)))
    return signal


@functools.lru_cache(maxsize=None)
def _make_kernel(n_rows, d_model, seq_len):
    assert n_rows % _NW == 0
    bpw = n_rows // _NW
    assert bpw % _CHUNK == 0
    steps = bpw // _CHUNK
    assert steps % _NBUF == 0 and steps > _NBUF
    assert d_model % 16 == 0

    mesh = plsc.VectorSubcoreMesh(core_axis_name="c", subcore_axis_name="s")

    @functools.partial(
        pl.kernel,
        out_type=jax.ShapeDtypeStruct((n_rows, d_model), jnp.float32),
        mesh=mesh,
        scratch_types=[
            pltpu.VMEM((bpw,), jnp.int32),
            pltpu.VMEM((2 * seq_len, d_model), jnp.float32),
        ]
        + [pltpu.VMEM((_CHUNK, d_model), jnp.float32)] * _NBUF
        + [pltpu.SemaphoreType.DMA] * (2 * _NBUF),
        compiler_params=pltpu.CompilerParams(use_tc_tiling_on_sc=False),
    )
    def k(emb_hbm, idx_hbm, pe_hbm, out_hbm, idx_v, pe_v, *bufs_and_sems):
        bufs = bufs_and_sems[:_NBUF]
        gsems = bufs_and_sems[_NBUF : 2 * _NBUF]
        osems = bufs_and_sems[2 * _NBUF :]

        wid = lax.axis_index("s") * _NC + lax.axis_index("c")
        base = wid * bpw
        pltpu.sync_copy(idx_hbm.at[pl.ds(base, bpw)], idx_v)
        pltpu.sync_copy(pe_hbm, pe_v)

        def gather(chunk, b):
            off = chunk * _CHUNK
            return pltpu.make_async_copy(
                emb_hbm.at[idx_v.at[pl.ds(off, _CHUNK)]], bufs[b], gsems[b]
            )

        def out_copy(chunk, b):
            off = chunk * _CHUNK
            return pltpu.make_async_copy(
                bufs[b], out_hbm.at[pl.ds(base + off, _CHUNK)], osems[b]
            )

        for b in range(_LOOKAHEAD):
            gather(b, b).start()

        @pl.loop(0, steps, step=_NBUF)
        def _step(s_):
            for b in range(_NBUF):
                chunk = s_ + b
                gather(chunk, b).wait()

                phase = lax.rem(chunk * _CHUNK, seq_len)
                buf = bufs[b]

                @pl.loop(0, _CHUNK, step=8)
                def _row(r0):
                    for dr in range(8):
                        r = r0 + dr
                        src = phase + r
                        for c4 in range(d_model // 16):
                            sl = pl.ds(c4 * 16, 16)
                            plsc.addupdate(buf.at[r, sl], pe_v[src, sl])

                out_copy(chunk, b).start()

                cn = chunk + _LOOKAHEAD
                bn = (b + _LOOKAHEAD) % _NBUF

                @pl.when(cn < steps)
                def _():
                    @pl.when(cn >= _NBUF)
                    def _():
                        out_copy(cn - _NBUF, bn).wait()

                    gather(cn, bn).start()

        for b in range(_NBUF):
            out_copy(steps - _NBUF + b, b).wait()

    return k


def kernel(x, emb_weight):
    batch, seq_len = x.shape
    _, d_model = emb_weight.shape
    pe = _pos_encoding(seq_len, d_model)
    pe2 = jnp.concatenate([pe, pe], axis=0)
    idx = x.reshape(-1).astype(jnp.int32)
    out = _make_kernel(batch * seq_len, d_model, seq_len)(
        emb_weight.astype(jnp.float32), idx, pe2
    )
    return out.reshape(batch, seq_len, d_model)
